# baseline (speedup 1.0000x reference)
import jax
import jax.numpy as jnp
from jax import lax
from jax.experimental import pallas as pl
from jax.experimental.pallas import tpu as pltpu
from jax.experimental.pallas import tpu_sc as plsc

B = 16384
BAG = 20
D = 64
L = 16

_INFO = plsc.get_sparse_core_info()
NC, NS = _INFO.num_cores, _INFO.num_subcores
NW = NC * NS
BPW = B // NW

C = 32
NCHUNK = BPW // C
IDX_PER_CHUNK = C * BAG
STREAM = 128
NSTREAM = IDX_PER_CHUNK // STREAM


def _bag_sum_body(xf_hbm, table_hbm, out_hbm, idx_v, rows_v, outb, sems):
    wid = lax.axis_index("s") * NC + lax.axis_index("c")
    pltpu.sync_copy(
        xf_hbm.at[pl.ds(wid * (BPW * BAG), BPW * BAG)], idx_v)

    def start_gathers(g, buf):
        return [
            pltpu.async_copy(
                table_hbm.at[idx_v.at[pl.ds(g * IDX_PER_CHUNK + j * STREAM, STREAM)]],
                rows_v.at[buf].at[pl.ds(j * STREAM, STREAM)],
                sems.at[buf],
            )
            for j in range(NSTREAM)
        ]

    def wait_gathers(buf):
        for j in range(NSTREAM):
            pltpu.make_async_copy(
                table_hbm.at[idx_v.at[pl.ds(j * STREAM, STREAM)]],
                rows_v.at[buf].at[pl.ds(j * STREAM, STREAM)],
                sems.at[buf],
            ).wait()

    def compute_chunk(g, buf):
        rv = rows_v.at[buf]

        def bag_body(b2, c2):
            r0 = b2 * (2 * BAG)
            accs = [
                rv[r0 + b_off * BAG, pl.ds(L * j, L)]
                for b_off in range(2)
                for j in range(D // L)
            ]
            for l in range(1, BAG):
                for k, (b_off, j) in enumerate(
                    (b, j) for b in range(2) for j in range(D // L)
                ):
                    accs[k] = accs[k] + rv[r0 + b_off * BAG + l, pl.ds(L * j, L)]
            for k, (b_off, j) in enumerate(
                (b, j) for b in range(2) for j in range(D // L)
            ):
                outb[2 * b2 + b_off, pl.ds(L * j, L)] = accs[k]
            return c2

        lax.fori_loop(0, C // 2, bag_body, 0)
        pltpu.sync_copy(outb, out_hbm.at[pl.ds(wid * BPW + g * C, C)])

    start_gathers(0, 0)

    def pair_body(g2, carry):
        g0 = 2 * g2
        start_gathers(g0 + 1, 1)
        wait_gathers(0)
        compute_chunk(g0, 0)
        start_gathers(g0 + 2, 0)
        wait_gathers(1)
        compute_chunk(g0 + 1, 1)
        return carry

    lax.fori_loop(0, NCHUNK // 2 - 1, pair_body, 0)
    start_gathers(NCHUNK - 1, 1)
    wait_gathers(0)
    compute_chunk(NCHUNK - 2, 0)
    wait_gathers(1)
    compute_chunk(NCHUNK - 1, 1)


@jax.jit
def _bag_sum(x, table):
    mesh = plsc.VectorSubcoreMesh(core_axis_name="c", subcore_axis_name="s")
    return pl.kernel(
        _bag_sum_body,
        out_type=jax.ShapeDtypeStruct((B, D), jnp.float32),
        mesh=mesh,
        scratch_types=[
            pltpu.VMEM((BPW * BAG,), jnp.int32),
            pltpu.VMEM((2, IDX_PER_CHUNK, D), jnp.float32),
            pltpu.VMEM((C, D), jnp.float32),
            pltpu.SemaphoreType.DMA((2,)),
        ],
        compiler_params=pltpu.CompilerParams(use_tc_tiling_on_sc=False),
    )(x, table)


def kernel(X, table):
    return _bag_sum(X.astype(jnp.int32).reshape(B * BAG), table)
